# baseline (speedup 1.0000x reference)
import functools

import jax
import jax.numpy as jnp
from jax import lax
from jax.experimental import pallas as pl
from jax.experimental.pallas import tpu as pltpu
from jax.experimental.pallas import tpu_sc as plsc

N_USERS = 5000
N_ITEMS = 5000
N = 10000
E = 320000
D = 128
NB = 3
T3 = NB * N
DUMP = T3
ACC_ROWS = 30720
EP = 327680
EROWS = EP // 128
BN_SCALE = float(1.0 / (1.0 + 1e-5) ** 0.5)

def _vmesh():
    return plsc.VectorSubcoreMesh(core_axis_name="c", subcore_axis_name="s",
                                  num_cores=2, num_subcores=16)


CHK = 5
NBLK = (EROWS // 16) // CHK


def _sc_segsum_body(table_hbm, gidx_hbm, sdst_hbm, zeros_hbm, out_hbm,
                    gi_v, di_v, rows_v, zb_v, acc_sh,
                    sem_g0, sem_g1, sem_i0, sem_i1):
    c = lax.axis_index("c")
    s = lax.axis_index("s")

    pltpu.sync_copy(zeros_hbm, zb_v)
    zbase = s * (ACC_ROWS // 16)
    ebase = s * (EROWS // 16)
    obase = s * 1872
    sems_g = (sem_g0, sem_g1)
    sems_i = (sem_i0, sem_i1)

    for q in range(2):
        qq = 2 * c + q

        @pl.loop(0, ACC_ROWS // 16, step=128)
        def _(r):
            pltpu.sync_copy(zb_v, acc_sh.at[pl.ds(zbase + r, 128)])

        plsc.subcore_barrier()

        def idx_load(g, p):
            blk = ebase + g * CHK
            pltpu.async_copy(gidx_hbm.at[qq, pl.ds(blk, CHK)], gi_v.at[p],
                             sems_i[p])
            pltpu.async_copy(sdst_hbm.at[pl.ds(blk, CHK)], di_v.at[p],
                             sems_i[p])

        def idx_wait(p):
            pltpu.make_async_copy(gidx_hbm.at[0, pl.ds(0, CHK)], gi_v.at[p],
                                  sems_i[p]).wait()
            pltpu.make_async_copy(sdst_hbm.at[pl.ds(0, CHK)], di_v.at[p],
                                  sems_i[p]).wait()

        def fire(p):
            for j in range(CHK):
                pltpu.async_copy(table_hbm.at[gi_v.at[p, j]], rows_v.at[p, j],
                                 sems_g[p])

        def drain(p):
            for j in range(CHK):
                pltpu.make_async_copy(table_hbm.at[pl.ds(0, 128)],
                                      rows_v.at[p, j], sems_g[p]).wait()

        def scatter(p):
            for j in range(CHK):
                pltpu.sync_copy(rows_v.at[p, j], acc_sh.at[di_v.at[p, j]],
                                add=True)

        idx_load(0, 0)
        idx_wait(0)
        fire(0)
        idx_load(1, 1)

        @pl.loop(0, NBLK // 2)
        def _(it):
            for p in (0, 1):
                g = 2 * it + p
                drain(p)
                if p == 0:
                    idx_wait(1)
                    fire(1)
                else:
                    @pl.when(it < NBLK // 2 - 1)
                    def _():
                        idx_wait(0)
                        fire(0)

                scatter(p)

                @pl.when(g + 2 < NBLK)
                def _():
                    idx_load(g + 2, p)

        plsc.subcore_barrier()

        pltpu.sync_copy(acc_sh.at[pl.ds(obase, 1872)],
                        out_hbm.at[qq, pl.ds(obase, 1872)])

        @pl.when(s == 0)
        def _():
            pltpu.sync_copy(acc_sh.at[pl.ds(16 * 1872, T3 - 16 * 1872)],
                            out_hbm.at[qq, pl.ds(16 * 1872, T3 - 16 * 1872)])

        plsc.subcore_barrier()


def _sc_segsum(table4, gidx_q, sdst, zeros_tile):
    kern = pl.kernel(
        _sc_segsum_body,
        out_type=jax.ShapeDtypeStruct((4, T3, 32), jnp.float32),
        mesh=_vmesh(),
        scratch_types=[
            pltpu.VMEM((2, CHK, 128), jnp.int32),
            pltpu.VMEM((2, CHK, 128), jnp.int32),
            pltpu.VMEM((2, CHK, 128, 32), jnp.float32),
            pltpu.VMEM((128, 32), jnp.float32),
            pltpu.VMEM_SHARED((ACC_ROWS, 32), jnp.float32),
            pltpu.SemaphoreType.DMA,
            pltpu.SemaphoreType.DMA,
            pltpu.SemaphoreType.DMA,
            pltpu.SemaphoreType.DMA,
        ],
        compiler_params=pltpu.CompilerParams(use_tc_tiling_on_sc=False),
    )
    return kern(table4, gidx_q, sdst, zeros_tile)


def _sc_count_body(sdst_hbm, ones_hbm, zeros_hbm, out_hbm,
                   di_v, ones_v, zb_v, cnt_sh):
    c = lax.axis_index("c")
    s = lax.axis_index("s")

    pltpu.sync_copy(zeros_hbm, zb_v)
    pltpu.sync_copy(ones_hbm, ones_v)
    zbase = s * (ACC_ROWS // 16)

    @pl.loop(0, ACC_ROWS // 16, step=128)
    def _(r):
        pltpu.sync_copy(zb_v, cnt_sh.at[pl.ds(zbase + r, 128)])

    plsc.subcore_barrier()

    ebase = c * (EROWS // 2) + s * (EROWS // 32)

    @pl.loop(0, EROWS // 32, step=8)
    def _(rb):
        base = ebase + rb
        pltpu.sync_copy(sdst_hbm.at[pl.ds(base, 8)], di_v)
        for j in range(8):
            pltpu.sync_copy(ones_v, cnt_sh.at[di_v.at[j]], add=True)

    plsc.subcore_barrier()

    obase = s * 1872
    pltpu.sync_copy(cnt_sh.at[pl.ds(obase, 1872)],
                    out_hbm.at[c, pl.ds(obase, 1872)])

    @pl.when(s == 0)
    def _():
        pltpu.sync_copy(cnt_sh.at[pl.ds(16 * 1872, T3 - 16 * 1872)],
                        out_hbm.at[c, pl.ds(16 * 1872, T3 - 16 * 1872)])


def _sc_count(sdst, ones_tile, zeros_tile16):
    kern = pl.kernel(
        _sc_count_body,
        out_type=jax.ShapeDtypeStruct((2, T3, 16), jnp.float32),
        mesh=_vmesh(),
        scratch_types=[
            pltpu.VMEM((8, 128), jnp.int32),
            pltpu.VMEM((128, 16), jnp.float32),
            pltpu.VMEM((128, 16), jnp.float32),
            pltpu.VMEM_SHARED((ACC_ROWS, 16), jnp.float32),
        ],
        compiler_params=pltpu.CompilerParams(use_tc_tiling_on_sc=False),
    )
    return kern(sdst, ones_tile, zeros_tile16)


BLK = 1000


def _proj_body(emb_ref, w_ref, b_ref, o_ref):
    e = emb_ref[...]
    w = w_ref[0]
    o_ref[...] = lax.dot_general(
        e, w, (((1,), (1,)), ((), ())),
        preferred_element_type=jnp.float32) + b_ref[0, 0][None, :]


def _tc_proj(emb_all, w_io, b_io):
    return pl.pallas_call(
        _proj_body,
        grid=(N // BLK,),
        in_specs=[
            pl.BlockSpec((BLK, D), lambda i: (i, 0)),
            pl.BlockSpec((1, D, D), lambda i: (i // (N_USERS // BLK), 0, 0)),
            pl.BlockSpec((1, 1, D), lambda i: (i // (N_USERS // BLK), 0, 0)),
        ],
        out_specs=pl.BlockSpec((BLK, D), lambda i: (i, 0)),
        out_shape=jax.ShapeDtypeStruct((N, D), jnp.float32),
    )(emb_all, w_io, b_io.reshape(2, 1, D))


def _quarter_matmul(agg_ref, b_idx, wl):
    za = lax.dot_general(agg_ref[0, b_idx], wl[:, 0:32],
                         (((1,), (1,)), ((), ())),
                         preferred_element_type=jnp.float32)
    for qq in range(1, 4):
        za += lax.dot_general(agg_ref[qq, b_idx], wl[:, 32 * qq:32 * (qq + 1)],
                              (((1,), (1,)), ((), ())),
                              preferred_element_type=jnp.float32)
    return za


def _pre_body(h_ref, wr_ref, o_ref):
    h = h_ref[...] if len(h_ref.shape) == 2 else h_ref[0]
    o_ref[0] = lax.dot_general(h, wr_ref[0], (((1,), (1,)), ((), ())),
                               preferred_element_type=jnp.float32)


def _tc_pre(h_in, wr):
    if h_in.ndim == 2:
        h_spec = pl.BlockSpec((BLK, D), lambda b, i: (i, 0))
    else:
        h_spec = pl.BlockSpec((1, BLK, D), lambda b, i: (b, i, 0))
    return pl.pallas_call(
        _pre_body,
        grid=(NB, N // BLK),
        in_specs=[h_spec,
                  pl.BlockSpec((1, D, D), lambda b, i: (b, 0, 0))],
        out_specs=pl.BlockSpec((1, BLK, D), lambda b, i: (b, i, 0)),
        out_shape=jax.ShapeDtypeStruct((NB, N, D), jnp.float32),
    )(h_in, wr)


def _post_body(agg_ref, cnt_ref, zr_ref, wl_ref, bl_ref,
               g_ref, be_ref, o_ref, *, relu):
    cnt = cnt_ref[0, 0, :, 0] + cnt_ref[1, 0, :, 0]
    cnt = jnp.maximum(cnt, 1.0)
    za = _quarter_matmul(agg_ref, 0, wl_ref[0])
    z = za / cnt[:, None] + bl_ref[0, 0][None, :] + zr_ref[0]
    z = g_ref[0, 0][None, :] * z * BN_SCALE + be_ref[0, 0][None, :]
    if relu:
        z = jnp.maximum(z, 0.0)
    o_ref[0] = z


def _tc_post(agg, cnt_part, zr, wl, bl, gamma, beta, relu):
    return pl.pallas_call(
        functools.partial(_post_body, relu=relu),
        grid=(NB, N // BLK),
        in_specs=[
            pl.BlockSpec((4, 1, BLK, 32), lambda b, i: (0, b, i, 0)),
            pl.BlockSpec((2, 1, BLK, 16), lambda b, i: (0, b, i, 0)),
            pl.BlockSpec((1, BLK, D), lambda b, i: (b, i, 0)),
            pl.BlockSpec((1, D, D), lambda b, i: (b, 0, 0)),
            pl.BlockSpec((1, 1, D), lambda b, i: (b, 0, 0)),
            pl.BlockSpec((1, 1, D), lambda b, i: (b, 0, 0)),
            pl.BlockSpec((1, 1, D), lambda b, i: (b, 0, 0)),
        ],
        out_specs=pl.BlockSpec((1, BLK, D), lambda b, i: (b, i, 0)),
        out_shape=jax.ShapeDtypeStruct((NB, N, D), jnp.float32),
    )(agg, cnt_part, zr, wl, bl.reshape(NB, 1, D),
      gamma.reshape(NB, 1, D), beta.reshape(NB, 1, D))


def _fuse_body(x_ref, agg_ref, cnt_ref, zr_ref, wl_ref, bl_ref, g_ref,
               be_ref, wq_ref, bq_ref, wk_ref, bk_ref,
               wf_ref, bf_ref, wr_ref, br_ref, o_ref):
    x = x_ref[...]
    q = lax.dot_general(x, wq_ref[...], (((1,), (1,)), ((), ())),
                        preferred_element_type=jnp.float32) + bq_ref[0][None, :]
    outs = []
    logits = []
    for b in range(NB):
        cnt = jnp.maximum(cnt_ref[0, b, :, 0] + cnt_ref[1, b, :, 0], 1.0)
        za = _quarter_matmul(agg_ref, b, wl_ref[b])
        z = za / cnt[:, None] + bl_ref[b, 0][None, :] + zr_ref[b]
        h2 = g_ref[b, 0][None, :] * z * BN_SCALE + be_ref[b, 0][None, :]
        ob = x + h2
        kb = lax.dot_general(ob, wk_ref[b], (((1,), (1,)), ((), ())),
                             preferred_element_type=jnp.float32) + bk_ref[b][None, :]
        outs.append(ob)
        logits.append(jnp.sum(q * kb, axis=-1))
    m = jnp.maximum(jnp.maximum(logits[0], logits[1]), logits[2])
    es = [jnp.exp(l - m) for l in logits]
    den = es[0] + es[1] + es[2]
    fused = (es[0][:, None] * outs[0] + es[1][:, None] * outs[1]
             + es[2][:, None] * outs[2]) / den[:, None]
    f = lax.dot_general(fused, wf_ref[...], (((1,), (1,)), ((), ())),
                        preferred_element_type=jnp.float32) + bf_ref[0][None, :]
    r = lax.dot_general(f, wr_ref[...], (((1,), (1,)), ((), ())),
                        preferred_element_type=jnp.float32) + br_ref[0][None, :]
    o_ref[...] = jnp.maximum(r, 0.0)


def _tc_fuse(x, agg, cnt_part, zr, wl, bl, gamma, beta,
             wq, bq, wk, bk, wf, bf, wr, br):
    return pl.pallas_call(
        _fuse_body,
        grid=(N // BLK,),
        in_specs=[
            pl.BlockSpec((BLK, D), lambda i: (i, 0)),
            pl.BlockSpec((4, NB, BLK, 32), lambda i: (0, 0, i, 0)),
            pl.BlockSpec((2, NB, BLK, 16), lambda i: (0, 0, i, 0)),
            pl.BlockSpec((NB, BLK, D), lambda i: (0, i, 0)),
            pl.BlockSpec((NB, D, D), lambda i: (0, 0, 0)),
            pl.BlockSpec((NB, 1, D), lambda i: (0, 0, 0)),
            pl.BlockSpec((NB, 1, D), lambda i: (0, 0, 0)),
            pl.BlockSpec((NB, 1, D), lambda i: (0, 0, 0)),
            pl.BlockSpec((D, D), lambda i: (0, 0)),
            pl.BlockSpec((1, D), lambda i: (0, 0)),
            pl.BlockSpec((NB, D, D), lambda i: (0, 0, 0)),
            pl.BlockSpec((NB, D), lambda i: (0, 0)),
            pl.BlockSpec((D, D), lambda i: (0, 0)),
            pl.BlockSpec((1, D), lambda i: (0, 0)),
            pl.BlockSpec((D, D), lambda i: (0, 0)),
            pl.BlockSpec((1, D), lambda i: (0, 0)),
        ],
        out_specs=pl.BlockSpec((BLK, D), lambda i: (i, 0)),
        out_shape=jax.ShapeDtypeStruct((N, D), jnp.float32),
    )(x, agg, cnt_part, zr, wl, bl.reshape(NB, 1, D),
      gamma.reshape(NB, 1, D), beta.reshape(NB, 1, D),
      wq, bq, wk, bk, wf, bf, wr, br)


def kernel(item_feats, edge_index, edge_type, params):
    src = edge_index[0].astype(jnp.int32)
    dst = edge_index[1].astype(jnp.int32)
    t = edge_type.astype(jnp.int32)

    pad = EP - E
    zpad = jnp.zeros((pad,), jnp.int32)
    g0 = jnp.concatenate([4 * src, zpad])
    g1 = jnp.concatenate([4 * (t * N + src), zpad])
    sd = jnp.concatenate([t * N + dst, jnp.full((pad,), DUMP, jnp.int32)])
    g0_q = jnp.stack([g0, g0 + 1, g0 + 2, g0 + 3]).reshape(4, EROWS, 128)
    g1_q = jnp.stack([g1, g1 + 1, g1 + 2, g1 + 3]).reshape(4, EROWS, 128)
    sd = sd.reshape(EROWS, 128)

    zeros32 = jnp.zeros((128, 32), jnp.float32)
    zeros16 = jnp.zeros((128, 16), jnp.float32)
    ones16 = jnp.ones((128, 16), jnp.float32)

    p = params
    emb_all = jnp.concatenate([p['user_emb'], item_feats], axis=0)
    w_io = jnp.stack([p['user_proj_W'], p['item_proj_W']])
    b_io = jnp.stack([p['user_proj_b'], p['item_proj_b']])

    wl = [jnp.stack([p['block%d_layer%d' % (b, l)]['Wl'] for b in range(NB)])
          for l in range(2)]
    bl = [jnp.stack([p['block%d_layer%d' % (b, l)]['bl'] for b in range(NB)])
          for l in range(2)]
    wr = [jnp.stack([p['block%d_layer%d' % (b, l)]['Wr'] for b in range(NB)])
          for l in range(2)]
    gm = [jnp.stack([p['block%d_layer%d' % (b, l)]['bn_gamma'] for b in range(NB)])
          for l in range(2)]
    bt = [jnp.stack([p['block%d_layer%d' % (b, l)]['bn_beta'] for b in range(NB)])
          for l in range(2)]
    wk = jnp.stack([p['key_proj%d_W' % b] for b in range(NB)])
    bk = jnp.stack([p['key_proj%d_b' % b] for b in range(NB)])

    x = _tc_proj(emb_all, w_io, b_io)

    cnt_part = _sc_count(sd, ones16, zeros16)
    cnt4 = cnt_part.reshape(2, NB, N, 16)

    agg0h = _sc_segsum(x.reshape(4 * N, 32), g0_q, sd, zeros32)
    zr0 = _tc_pre(x, wr[0])
    h1 = _tc_post(agg0h.reshape(4, NB, N, 32), cnt4, zr0,
                  wl[0], bl[0], gm[0], bt[0], True)

    agg1h = _sc_segsum(h1.reshape(4 * T3, 32), g1_q, sd, zeros32)
    zr1 = _tc_pre(h1, wr[1])

    return _tc_fuse(x, agg1h.reshape(4, NB, N, 32), cnt4, zr1,
                    wl[1], bl[1], gm[1], bt[1],
                    p['query_proj_W'], p['query_proj_b'].reshape(1, D),
                    wk, bk, p['fuse_W'], p['fuse_b'].reshape(1, D),
                    p['refine_W'], p['refine_b'].reshape(1, D))


# speedup vs baseline: 1.0050x; 1.0050x over previous
import functools

import jax
import jax.numpy as jnp
from jax import lax
from jax.experimental import pallas as pl
from jax.experimental.pallas import tpu as pltpu
from jax.experimental.pallas import tpu_sc as plsc

N_USERS = 5000
N_ITEMS = 5000
N = 10000
E = 320000
D = 128
NB = 3
T3 = NB * N
DUMP = T3
ACC_ROWS = 30720
EP = 327680
EROWS = EP // 128
BN_SCALE = float(1.0 / (1.0 + 1e-5) ** 0.5)

def _vmesh():
    return plsc.VectorSubcoreMesh(core_axis_name="c", subcore_axis_name="s",
                                  num_cores=2, num_subcores=16)


CHK = 5
NBLK = (EROWS // 16) // CHK


def _sc_segsum_body(table_hbm, gidx_hbm, sdst_hbm, zeros_hbm, out_hbm,
                    gi_v, di_v, rows_v, zb_v, acc_sh,
                    sem_g0, sem_g1, sem_i0, sem_i1):
    c = lax.axis_index("c")
    s = lax.axis_index("s")

    pltpu.sync_copy(zeros_hbm, zb_v)
    zbase = s * (ACC_ROWS // 16)
    ebase = s * (EROWS // 16)
    obase = s * 1872
    sems_g = (sem_g0, sem_g1)
    sems_i = (sem_i0, sem_i1)

    for q in range(2):
        qq = 2 * c + q

        @pl.loop(0, ACC_ROWS // 16, step=128)
        def _(r):
            pltpu.sync_copy(zb_v, acc_sh.at[pl.ds(zbase + r, 128)])

        plsc.subcore_barrier()

        def idx_load(g, p):
            blk = ebase + g * CHK
            pltpu.async_copy(gidx_hbm.at[qq, pl.ds(blk, CHK)], gi_v.at[p],
                             sems_i[p])
            pltpu.async_copy(sdst_hbm.at[pl.ds(blk, CHK)], di_v.at[p],
                             sems_i[p])

        def idx_wait(p):
            pltpu.make_async_copy(gidx_hbm.at[0, pl.ds(0, CHK)], gi_v.at[p],
                                  sems_i[p]).wait()
            pltpu.make_async_copy(sdst_hbm.at[pl.ds(0, CHK)], di_v.at[p],
                                  sems_i[p]).wait()

        def fire(p):
            for j in range(CHK):
                pltpu.async_copy(table_hbm.at[gi_v.at[p, j]], rows_v.at[p, j],
                                 sems_g[p])

        def drain(p):
            for j in range(CHK):
                pltpu.make_async_copy(table_hbm.at[pl.ds(0, 128)],
                                      rows_v.at[p, j], sems_g[p]).wait()

        def scatter(p):
            for j in range(CHK):
                pltpu.sync_copy(rows_v.at[p, j], acc_sh.at[di_v.at[p, j]],
                                add=True)

        idx_load(0, 0)
        idx_wait(0)
        fire(0)
        idx_load(1, 1)

        @pl.loop(0, NBLK // 2)
        def _(it):
            for p in (0, 1):
                g = 2 * it + p
                drain(p)
                if p == 0:
                    idx_wait(1)
                    fire(1)
                else:
                    @pl.when(it < NBLK // 2 - 1)
                    def _():
                        idx_wait(0)
                        fire(0)

                scatter(p)

                @pl.when(g + 2 < NBLK)
                def _():
                    idx_load(g + 2, p)

        plsc.subcore_barrier()

        pltpu.sync_copy(acc_sh.at[pl.ds(obase, 1872)],
                        out_hbm.at[qq, pl.ds(obase, 1872)])

        @pl.when(s == 0)
        def _():
            pltpu.sync_copy(acc_sh.at[pl.ds(16 * 1872, T3 - 16 * 1872)],
                            out_hbm.at[qq, pl.ds(16 * 1872, T3 - 16 * 1872)])

        plsc.subcore_barrier()


def _sc_segsum(table4, gidx_q, sdst, zeros_tile):
    kern = pl.kernel(
        _sc_segsum_body,
        out_type=jax.ShapeDtypeStruct((4, T3, 32), jnp.float32),
        mesh=_vmesh(),
        scratch_types=[
            pltpu.VMEM((2, CHK, 128), jnp.int32),
            pltpu.VMEM((2, CHK, 128), jnp.int32),
            pltpu.VMEM((2, CHK, 128, 32), jnp.float32),
            pltpu.VMEM((128, 32), jnp.float32),
            pltpu.VMEM_SHARED((ACC_ROWS, 32), jnp.float32),
            pltpu.SemaphoreType.DMA,
            pltpu.SemaphoreType.DMA,
            pltpu.SemaphoreType.DMA,
            pltpu.SemaphoreType.DMA,
        ],
        compiler_params=pltpu.CompilerParams(use_tc_tiling_on_sc=False),
    )
    return kern(table4, gidx_q, sdst, zeros_tile)


def _sc_count_body(sdst_hbm, ones_hbm, zeros_hbm, out_hbm,
                   di_v, ones_v, zb_v, cnt_sh):
    c = lax.axis_index("c")
    s = lax.axis_index("s")

    pltpu.sync_copy(zeros_hbm, zb_v)
    pltpu.sync_copy(ones_hbm, ones_v)
    zbase = s * (ACC_ROWS // 16)

    @pl.loop(0, ACC_ROWS // 16, step=128)
    def _(r):
        pltpu.sync_copy(zb_v, cnt_sh.at[pl.ds(zbase + r, 128)])

    plsc.subcore_barrier()

    ebase = c * (EROWS // 2) + s * (EROWS // 32)

    @pl.loop(0, EROWS // 32, step=8)
    def _(rb):
        base = ebase + rb
        pltpu.sync_copy(sdst_hbm.at[pl.ds(base, 8)], di_v)
        for j in range(8):
            pltpu.sync_copy(ones_v, cnt_sh.at[di_v.at[j]], add=True)

    plsc.subcore_barrier()

    obase = s * 1872
    pltpu.sync_copy(cnt_sh.at[pl.ds(obase, 1872)],
                    out_hbm.at[c, pl.ds(obase, 1872)])

    @pl.when(s == 0)
    def _():
        pltpu.sync_copy(cnt_sh.at[pl.ds(16 * 1872, T3 - 16 * 1872)],
                        out_hbm.at[c, pl.ds(16 * 1872, T3 - 16 * 1872)])


def _sc_count(sdst, ones_tile, zeros_tile16):
    kern = pl.kernel(
        _sc_count_body,
        out_type=jax.ShapeDtypeStruct((2, T3, 16), jnp.float32),
        mesh=_vmesh(),
        scratch_types=[
            pltpu.VMEM((8, 128), jnp.int32),
            pltpu.VMEM((128, 16), jnp.float32),
            pltpu.VMEM((128, 16), jnp.float32),
            pltpu.VMEM_SHARED((ACC_ROWS, 16), jnp.float32),
        ],
        compiler_params=pltpu.CompilerParams(use_tc_tiling_on_sc=False),
    )
    return kern(sdst, ones_tile, zeros_tile16)


BLK = 1000


def _proj_body(emb_ref, w_ref, b_ref, o_ref):
    e = emb_ref[...]
    w = w_ref[0]
    o_ref[...] = lax.dot_general(
        e, w, (((1,), (1,)), ((), ())),
        preferred_element_type=jnp.float32) + b_ref[0, 0][None, :]


def _tc_proj(emb_all, w_io, b_io):
    return pl.pallas_call(
        _proj_body,
        grid=(N // BLK,),
        in_specs=[
            pl.BlockSpec((BLK, D), lambda i: (i, 0)),
            pl.BlockSpec((1, D, D), lambda i: (i // (N_USERS // BLK), 0, 0)),
            pl.BlockSpec((1, 1, D), lambda i: (i // (N_USERS // BLK), 0, 0)),
        ],
        out_specs=pl.BlockSpec((BLK, D), lambda i: (i, 0)),
        out_shape=jax.ShapeDtypeStruct((N, D), jnp.float32),
    )(emb_all, w_io, b_io.reshape(2, 1, D))


def _quarter_matmul(agg_ref, b_idx, wl):
    za = lax.dot_general(agg_ref[0, b_idx], wl[:, 0:32],
                         (((1,), (1,)), ((), ())),
                         preferred_element_type=jnp.float32)
    for qq in range(1, 4):
        za += lax.dot_general(agg_ref[qq, b_idx], wl[:, 32 * qq:32 * (qq + 1)],
                              (((1,), (1,)), ((), ())),
                              preferred_element_type=jnp.float32)
    return za


def _post_body(agg_ref, cnt_ref, h_ref, wl_ref, bl_ref, wr_ref,
               g_ref, be_ref, o_ref, *, relu):
    h = h_ref[...] if len(h_ref.shape) == 2 else h_ref[0]
    cnt = cnt_ref[0, 0, :, 0] + cnt_ref[1, 0, :, 0]
    cnt = jnp.maximum(cnt, 1.0)
    za = _quarter_matmul(agg_ref, 0, wl_ref[0])
    z = (za / cnt[:, None] + bl_ref[0, 0][None, :]
         + lax.dot_general(h, wr_ref[0], (((1,), (1,)), ((), ())),
                           preferred_element_type=jnp.float32))
    z = g_ref[0, 0][None, :] * z * BN_SCALE + be_ref[0, 0][None, :]
    if relu:
        z = jnp.maximum(z, 0.0)
    o_ref[0] = z


def _tc_post(agg, cnt_part, h_in, wl, bl, wr, gamma, beta, relu):
    if h_in.ndim == 2:
        h_spec = pl.BlockSpec((BLK, D), lambda b, i: (i, 0))
    else:
        h_spec = pl.BlockSpec((1, BLK, D), lambda b, i: (b, i, 0))
    return pl.pallas_call(
        functools.partial(_post_body, relu=relu),
        grid=(NB, N // BLK),
        in_specs=[
            pl.BlockSpec((4, 1, BLK, 32), lambda b, i: (0, b, i, 0)),
            pl.BlockSpec((2, 1, BLK, 16), lambda b, i: (0, b, i, 0)),
            h_spec,
            pl.BlockSpec((1, D, D), lambda b, i: (b, 0, 0)),
            pl.BlockSpec((1, 1, D), lambda b, i: (b, 0, 0)),
            pl.BlockSpec((1, D, D), lambda b, i: (b, 0, 0)),
            pl.BlockSpec((1, 1, D), lambda b, i: (b, 0, 0)),
            pl.BlockSpec((1, 1, D), lambda b, i: (b, 0, 0)),
        ],
        out_specs=pl.BlockSpec((1, BLK, D), lambda b, i: (b, i, 0)),
        out_shape=jax.ShapeDtypeStruct((NB, N, D), jnp.float32),
    )(agg, cnt_part, h_in, wl, bl.reshape(NB, 1, D), wr,
      gamma.reshape(NB, 1, D), beta.reshape(NB, 1, D))


def _fuse_body(x_ref, agg_ref, cnt_ref, h1_ref, wl_ref, bl_ref, wr1_ref,
               g_ref, be_ref, wq_ref, bq_ref, wk_ref, bk_ref,
               wf_ref, bf_ref, wr_ref, br_ref, o_ref):
    x = x_ref[...]
    q = lax.dot_general(x, wq_ref[...], (((1,), (1,)), ((), ())),
                        preferred_element_type=jnp.float32) + bq_ref[0][None, :]
    outs = []
    logits = []
    for b in range(NB):
        cnt = jnp.maximum(cnt_ref[0, b, :, 0] + cnt_ref[1, b, :, 0], 1.0)
        za = _quarter_matmul(agg_ref, b, wl_ref[b])
        zr = lax.dot_general(h1_ref[b], wr1_ref[b], (((1,), (1,)), ((), ())),
                             preferred_element_type=jnp.float32)
        z = za / cnt[:, None] + bl_ref[b, 0][None, :] + zr
        h2 = g_ref[b, 0][None, :] * z * BN_SCALE + be_ref[b, 0][None, :]
        ob = x + h2
        kb = lax.dot_general(ob, wk_ref[b], (((1,), (1,)), ((), ())),
                             preferred_element_type=jnp.float32) + bk_ref[b][None, :]
        outs.append(ob)
        logits.append(jnp.sum(q * kb, axis=-1))
    m = jnp.maximum(jnp.maximum(logits[0], logits[1]), logits[2])
    es = [jnp.exp(l - m) for l in logits]
    den = es[0] + es[1] + es[2]
    fused = (es[0][:, None] * outs[0] + es[1][:, None] * outs[1]
             + es[2][:, None] * outs[2]) / den[:, None]
    f = lax.dot_general(fused, wf_ref[...], (((1,), (1,)), ((), ())),
                        preferred_element_type=jnp.float32) + bf_ref[0][None, :]
    r = lax.dot_general(f, wr_ref[...], (((1,), (1,)), ((), ())),
                        preferred_element_type=jnp.float32) + br_ref[0][None, :]
    o_ref[...] = jnp.maximum(r, 0.0)


def _tc_fuse(x, agg, cnt_part, h1, wl, bl, wr1, gamma, beta,
             wq, bq, wk, bk, wf, bf, wr, br):
    return pl.pallas_call(
        _fuse_body,
        grid=(N // BLK,),
        in_specs=[
            pl.BlockSpec((BLK, D), lambda i: (i, 0)),
            pl.BlockSpec((4, NB, BLK, 32), lambda i: (0, 0, i, 0)),
            pl.BlockSpec((2, NB, BLK, 16), lambda i: (0, 0, i, 0)),
            pl.BlockSpec((NB, BLK, D), lambda i: (0, i, 0)),
            pl.BlockSpec((NB, D, D), lambda i: (0, 0, 0)),
            pl.BlockSpec((NB, 1, D), lambda i: (0, 0, 0)),
            pl.BlockSpec((NB, D, D), lambda i: (0, 0, 0)),
            pl.BlockSpec((NB, 1, D), lambda i: (0, 0, 0)),
            pl.BlockSpec((NB, 1, D), lambda i: (0, 0, 0)),
            pl.BlockSpec((D, D), lambda i: (0, 0)),
            pl.BlockSpec((1, D), lambda i: (0, 0)),
            pl.BlockSpec((NB, D, D), lambda i: (0, 0, 0)),
            pl.BlockSpec((NB, D), lambda i: (0, 0)),
            pl.BlockSpec((D, D), lambda i: (0, 0)),
            pl.BlockSpec((1, D), lambda i: (0, 0)),
            pl.BlockSpec((D, D), lambda i: (0, 0)),
            pl.BlockSpec((1, D), lambda i: (0, 0)),
        ],
        out_specs=pl.BlockSpec((BLK, D), lambda i: (i, 0)),
        out_shape=jax.ShapeDtypeStruct((N, D), jnp.float32),
    )(x, agg, cnt_part, h1, wl, bl.reshape(NB, 1, D), wr1,
      gamma.reshape(NB, 1, D), beta.reshape(NB, 1, D),
      wq, bq, wk, bk, wf, bf, wr, br)


def kernel(item_feats, edge_index, edge_type, params):
    src = edge_index[0].astype(jnp.int32)
    dst = edge_index[1].astype(jnp.int32)
    t = edge_type.astype(jnp.int32)

    pad = EP - E
    zpad = jnp.zeros((pad,), jnp.int32)
    g0 = jnp.concatenate([4 * src, zpad])
    g1 = jnp.concatenate([4 * (t * N + src), zpad])
    sd = jnp.concatenate([t * N + dst, jnp.full((pad,), DUMP, jnp.int32)])
    g0_q = jnp.stack([g0, g0 + 1, g0 + 2, g0 + 3]).reshape(4, EROWS, 128)
    g1_q = jnp.stack([g1, g1 + 1, g1 + 2, g1 + 3]).reshape(4, EROWS, 128)
    sd = sd.reshape(EROWS, 128)

    zeros32 = jnp.zeros((128, 32), jnp.float32)
    zeros16 = jnp.zeros((128, 16), jnp.float32)
    ones16 = jnp.ones((128, 16), jnp.float32)

    p = params
    emb_all = jnp.concatenate([p['user_emb'], item_feats], axis=0)
    w_io = jnp.stack([p['user_proj_W'], p['item_proj_W']])
    b_io = jnp.stack([p['user_proj_b'], p['item_proj_b']])

    wl = [jnp.stack([p['block%d_layer%d' % (b, l)]['Wl'] for b in range(NB)])
          for l in range(2)]
    bl = [jnp.stack([p['block%d_layer%d' % (b, l)]['bl'] for b in range(NB)])
          for l in range(2)]
    wr = [jnp.stack([p['block%d_layer%d' % (b, l)]['Wr'] for b in range(NB)])
          for l in range(2)]
    gm = [jnp.stack([p['block%d_layer%d' % (b, l)]['bn_gamma'] for b in range(NB)])
          for l in range(2)]
    bt = [jnp.stack([p['block%d_layer%d' % (b, l)]['bn_beta'] for b in range(NB)])
          for l in range(2)]
    wk = jnp.stack([p['key_proj%d_W' % b] for b in range(NB)])
    bk = jnp.stack([p['key_proj%d_b' % b] for b in range(NB)])

    x = _tc_proj(emb_all, w_io, b_io)

    cnt_part = _sc_count(sd, ones16, zeros16)
    cnt4 = cnt_part.reshape(2, NB, N, 16)

    agg0h = _sc_segsum(x.reshape(4 * N, 32), g0_q, sd, zeros32)
    h1 = _tc_post(agg0h.reshape(4, NB, N, 32), cnt4, x,
                  wl[0], bl[0], wr[0], gm[0], bt[0], True)

    agg1h = _sc_segsum(h1.reshape(4 * T3, 32), g1_q, sd, zeros32)

    return _tc_fuse(x, agg1h.reshape(4, NB, N, 32), cnt4, h1,
                    wl[1], bl[1], wr[1], gm[1], bt[1],
                    p['query_proj_W'], p['query_proj_b'].reshape(1, D),
                    wk, bk, p['fuse_W'], p['fuse_b'].reshape(1, D),
                    p['refine_W'], p['refine_b'].reshape(1, D))


# speedup vs baseline: 1.0062x; 1.0012x over previous
import functools

import jax
import jax.numpy as jnp
from jax import lax
from jax.experimental import pallas as pl
from jax.experimental.pallas import tpu as pltpu
from jax.experimental.pallas import tpu_sc as plsc

N_USERS = 5000
N_ITEMS = 5000
N = 10000
E = 320000
D = 128
NB = 3
T3 = NB * N
DUMP = T3
ACC_ROWS = 30720
EP = 327680
EROWS = EP // 128
BN_SCALE = float(1.0 / (1.0 + 1e-5) ** 0.5)

def _vmesh():
    return plsc.VectorSubcoreMesh(core_axis_name="c", subcore_axis_name="s",
                                  num_cores=2, num_subcores=16)


CHK = 5
NBLK = (EROWS // 16) // CHK


def _sc_segsum_body(table_hbm, gidx_hbm, sdst_hbm, zeros_hbm, out_hbm,
                    gi_v, di_v, rows_v, zb_v, acc_sh,
                    sem_g0, sem_g1, sem_i0, sem_i1):
    c = lax.axis_index("c")
    s = lax.axis_index("s")

    pltpu.sync_copy(zeros_hbm, zb_v)
    zbase = s * (ACC_ROWS // 16)
    ebase = s * (EROWS // 16)
    obase = s * 1872
    sems_g = (sem_g0, sem_g1)
    sems_i = (sem_i0, sem_i1)

    for q in range(2):
        qq = 2 * c + q

        @pl.loop(0, ACC_ROWS // 16, step=128)
        def _(r):
            pltpu.sync_copy(zb_v, acc_sh.at[pl.ds(zbase + r, 128)])

        plsc.subcore_barrier()

        def idx_load(g, p):
            blk = ebase + g * CHK
            pltpu.async_copy(gidx_hbm.at[qq, pl.ds(blk, CHK)], gi_v.at[p],
                             sems_i[p])
            pltpu.async_copy(sdst_hbm.at[pl.ds(blk, CHK)], di_v.at[p],
                             sems_i[p])

        def idx_wait(p):
            pltpu.make_async_copy(gidx_hbm.at[0, pl.ds(0, CHK)], gi_v.at[p],
                                  sems_i[p]).wait()
            pltpu.make_async_copy(sdst_hbm.at[pl.ds(0, CHK)], di_v.at[p],
                                  sems_i[p]).wait()

        def fire(p):
            for j in range(CHK):
                pltpu.async_copy(table_hbm.at[gi_v.at[p, j]], rows_v.at[p, j],
                                 sems_g[p])

        def drain(p):
            for j in range(CHK):
                pltpu.make_async_copy(table_hbm.at[pl.ds(0, 128)],
                                      rows_v.at[p, j], sems_g[p]).wait()

        def scatter(p):
            for j in range(CHK):
                pltpu.sync_copy(rows_v.at[p, j], acc_sh.at[di_v.at[p, j]],
                                add=True)

        idx_load(0, 0)
        idx_wait(0)
        fire(0)
        idx_load(1, 1)

        @pl.loop(0, NBLK // 2)
        def _(it):
            for p in (0, 1):
                g = 2 * it + p
                if p == 0:
                    idx_wait(1)
                    fire(1)
                else:
                    @pl.when(it < NBLK // 2 - 1)
                    def _():
                        idx_wait(0)
                        fire(0)
                drain(p)
                scatter(p)

                @pl.when(g + 2 < NBLK)
                def _():
                    idx_load(g + 2, p)

        plsc.subcore_barrier()

        pltpu.sync_copy(acc_sh.at[pl.ds(obase, 1872)],
                        out_hbm.at[qq, pl.ds(obase, 1872)])

        @pl.when(s == 0)
        def _():
            pltpu.sync_copy(acc_sh.at[pl.ds(16 * 1872, T3 - 16 * 1872)],
                            out_hbm.at[qq, pl.ds(16 * 1872, T3 - 16 * 1872)])

        plsc.subcore_barrier()


def _sc_segsum(table4, gidx_q, sdst, zeros_tile):
    kern = pl.kernel(
        _sc_segsum_body,
        out_type=jax.ShapeDtypeStruct((4, T3, 32), jnp.float32),
        mesh=_vmesh(),
        scratch_types=[
            pltpu.VMEM((2, CHK, 128), jnp.int32),
            pltpu.VMEM((2, CHK, 128), jnp.int32),
            pltpu.VMEM((2, CHK, 128, 32), jnp.float32),
            pltpu.VMEM((128, 32), jnp.float32),
            pltpu.VMEM_SHARED((ACC_ROWS, 32), jnp.float32),
            pltpu.SemaphoreType.DMA,
            pltpu.SemaphoreType.DMA,
            pltpu.SemaphoreType.DMA,
            pltpu.SemaphoreType.DMA,
        ],
        compiler_params=pltpu.CompilerParams(use_tc_tiling_on_sc=False),
    )
    return kern(table4, gidx_q, sdst, zeros_tile)


def _sc_count_body(sdst_hbm, ones_hbm, zeros_hbm, out_hbm,
                   di_v, ones_v, zb_v, cnt_sh):
    c = lax.axis_index("c")
    s = lax.axis_index("s")

    pltpu.sync_copy(zeros_hbm, zb_v)
    pltpu.sync_copy(ones_hbm, ones_v)
    zbase = s * (ACC_ROWS // 16)

    @pl.loop(0, ACC_ROWS // 16, step=128)
    def _(r):
        pltpu.sync_copy(zb_v, cnt_sh.at[pl.ds(zbase + r, 128)])

    plsc.subcore_barrier()

    ebase = c * (EROWS // 2) + s * (EROWS // 32)

    @pl.loop(0, EROWS // 32, step=8)
    def _(rb):
        base = ebase + rb
        pltpu.sync_copy(sdst_hbm.at[pl.ds(base, 8)], di_v)
        for j in range(8):
            pltpu.sync_copy(ones_v, cnt_sh.at[di_v.at[j]], add=True)

    plsc.subcore_barrier()

    obase = s * 1872
    pltpu.sync_copy(cnt_sh.at[pl.ds(obase, 1872)],
                    out_hbm.at[c, pl.ds(obase, 1872)])

    @pl.when(s == 0)
    def _():
        pltpu.sync_copy(cnt_sh.at[pl.ds(16 * 1872, T3 - 16 * 1872)],
                        out_hbm.at[c, pl.ds(16 * 1872, T3 - 16 * 1872)])


def _sc_count(sdst, ones_tile, zeros_tile16):
    kern = pl.kernel(
        _sc_count_body,
        out_type=jax.ShapeDtypeStruct((2, T3, 16), jnp.float32),
        mesh=_vmesh(),
        scratch_types=[
            pltpu.VMEM((8, 128), jnp.int32),
            pltpu.VMEM((128, 16), jnp.float32),
            pltpu.VMEM((128, 16), jnp.float32),
            pltpu.VMEM_SHARED((ACC_ROWS, 16), jnp.float32),
        ],
        compiler_params=pltpu.CompilerParams(use_tc_tiling_on_sc=False),
    )
    return kern(sdst, ones_tile, zeros_tile16)


BLK = 1000


def _proj_body(emb_ref, w_ref, b_ref, o_ref):
    e = emb_ref[...]
    w = w_ref[0]
    o_ref[...] = lax.dot_general(
        e, w, (((1,), (1,)), ((), ())),
        preferred_element_type=jnp.float32) + b_ref[0, 0][None, :]


def _tc_proj(emb_all, w_io, b_io):
    return pl.pallas_call(
        _proj_body,
        grid=(N // BLK,),
        in_specs=[
            pl.BlockSpec((BLK, D), lambda i: (i, 0)),
            pl.BlockSpec((1, D, D), lambda i: (i // (N_USERS // BLK), 0, 0)),
            pl.BlockSpec((1, 1, D), lambda i: (i // (N_USERS // BLK), 0, 0)),
        ],
        out_specs=pl.BlockSpec((BLK, D), lambda i: (i, 0)),
        out_shape=jax.ShapeDtypeStruct((N, D), jnp.float32),
    )(emb_all, w_io, b_io.reshape(2, 1, D))


def _quarter_matmul(agg_ref, b_idx, wl):
    za = lax.dot_general(agg_ref[0, b_idx], wl[:, 0:32],
                         (((1,), (1,)), ((), ())),
                         preferred_element_type=jnp.float32)
    for qq in range(1, 4):
        za += lax.dot_general(agg_ref[qq, b_idx], wl[:, 32 * qq:32 * (qq + 1)],
                              (((1,), (1,)), ((), ())),
                              preferred_element_type=jnp.float32)
    return za


def _post_body(agg_ref, cnt_ref, h_ref, wl_ref, bl_ref, wr_ref,
               g_ref, be_ref, o_ref, *, relu):
    h = h_ref[...] if len(h_ref.shape) == 2 else h_ref[0]
    cnt = cnt_ref[0, 0, :, 0] + cnt_ref[1, 0, :, 0]
    cnt = jnp.maximum(cnt, 1.0)
    za = _quarter_matmul(agg_ref, 0, wl_ref[0])
    z = (za / cnt[:, None] + bl_ref[0, 0][None, :]
         + lax.dot_general(h, wr_ref[0], (((1,), (1,)), ((), ())),
                           preferred_element_type=jnp.float32))
    z = g_ref[0, 0][None, :] * z * BN_SCALE + be_ref[0, 0][None, :]
    if relu:
        z = jnp.maximum(z, 0.0)
    o_ref[0] = z


def _tc_post(agg, cnt_part, h_in, wl, bl, wr, gamma, beta, relu):
    if h_in.ndim == 2:
        h_spec = pl.BlockSpec((BLK, D), lambda b, i: (i, 0))
    else:
        h_spec = pl.BlockSpec((1, BLK, D), lambda b, i: (b, i, 0))
    return pl.pallas_call(
        functools.partial(_post_body, relu=relu),
        grid=(NB, N // BLK),
        in_specs=[
            pl.BlockSpec((4, 1, BLK, 32), lambda b, i: (0, b, i, 0)),
            pl.BlockSpec((2, 1, BLK, 16), lambda b, i: (0, b, i, 0)),
            h_spec,
            pl.BlockSpec((1, D, D), lambda b, i: (b, 0, 0)),
            pl.BlockSpec((1, 1, D), lambda b, i: (b, 0, 0)),
            pl.BlockSpec((1, D, D), lambda b, i: (b, 0, 0)),
            pl.BlockSpec((1, 1, D), lambda b, i: (b, 0, 0)),
            pl.BlockSpec((1, 1, D), lambda b, i: (b, 0, 0)),
        ],
        out_specs=pl.BlockSpec((1, BLK, D), lambda b, i: (b, i, 0)),
        out_shape=jax.ShapeDtypeStruct((NB, N, D), jnp.float32),
    )(agg, cnt_part, h_in, wl, bl.reshape(NB, 1, D), wr,
      gamma.reshape(NB, 1, D), beta.reshape(NB, 1, D))


def _fuse_body(x_ref, agg_ref, cnt_ref, h1_ref, wl_ref, bl_ref, wr1_ref,
               g_ref, be_ref, wq_ref, bq_ref, wk_ref, bk_ref,
               wf_ref, bf_ref, wr_ref, br_ref, o_ref):
    x = x_ref[...]
    q = lax.dot_general(x, wq_ref[...], (((1,), (1,)), ((), ())),
                        preferred_element_type=jnp.float32) + bq_ref[0][None, :]
    outs = []
    logits = []
    for b in range(NB):
        cnt = jnp.maximum(cnt_ref[0, b, :, 0] + cnt_ref[1, b, :, 0], 1.0)
        za = _quarter_matmul(agg_ref, b, wl_ref[b])
        zr = lax.dot_general(h1_ref[b], wr1_ref[b], (((1,), (1,)), ((), ())),
                             preferred_element_type=jnp.float32)
        z = za / cnt[:, None] + bl_ref[b, 0][None, :] + zr
        h2 = g_ref[b, 0][None, :] * z * BN_SCALE + be_ref[b, 0][None, :]
        ob = x + h2
        kb = lax.dot_general(ob, wk_ref[b], (((1,), (1,)), ((), ())),
                             preferred_element_type=jnp.float32) + bk_ref[b][None, :]
        outs.append(ob)
        logits.append(jnp.sum(q * kb, axis=-1))
    m = jnp.maximum(jnp.maximum(logits[0], logits[1]), logits[2])
    es = [jnp.exp(l - m) for l in logits]
    den = es[0] + es[1] + es[2]
    fused = (es[0][:, None] * outs[0] + es[1][:, None] * outs[1]
             + es[2][:, None] * outs[2]) / den[:, None]
    f = lax.dot_general(fused, wf_ref[...], (((1,), (1,)), ((), ())),
                        preferred_element_type=jnp.float32) + bf_ref[0][None, :]
    r = lax.dot_general(f, wr_ref[...], (((1,), (1,)), ((), ())),
                        preferred_element_type=jnp.float32) + br_ref[0][None, :]
    o_ref[...] = jnp.maximum(r, 0.0)


def _tc_fuse(x, agg, cnt_part, h1, wl, bl, wr1, gamma, beta,
             wq, bq, wk, bk, wf, bf, wr, br):
    return pl.pallas_call(
        _fuse_body,
        grid=(N // BLK,),
        in_specs=[
            pl.BlockSpec((BLK, D), lambda i: (i, 0)),
            pl.BlockSpec((4, NB, BLK, 32), lambda i: (0, 0, i, 0)),
            pl.BlockSpec((2, NB, BLK, 16), lambda i: (0, 0, i, 0)),
            pl.BlockSpec((NB, BLK, D), lambda i: (0, i, 0)),
            pl.BlockSpec((NB, D, D), lambda i: (0, 0, 0)),
            pl.BlockSpec((NB, 1, D), lambda i: (0, 0, 0)),
            pl.BlockSpec((NB, D, D), lambda i: (0, 0, 0)),
            pl.BlockSpec((NB, 1, D), lambda i: (0, 0, 0)),
            pl.BlockSpec((NB, 1, D), lambda i: (0, 0, 0)),
            pl.BlockSpec((D, D), lambda i: (0, 0)),
            pl.BlockSpec((1, D), lambda i: (0, 0)),
            pl.BlockSpec((NB, D, D), lambda i: (0, 0, 0)),
            pl.BlockSpec((NB, D), lambda i: (0, 0)),
            pl.BlockSpec((D, D), lambda i: (0, 0)),
            pl.BlockSpec((1, D), lambda i: (0, 0)),
            pl.BlockSpec((D, D), lambda i: (0, 0)),
            pl.BlockSpec((1, D), lambda i: (0, 0)),
        ],
        out_specs=pl.BlockSpec((BLK, D), lambda i: (i, 0)),
        out_shape=jax.ShapeDtypeStruct((N, D), jnp.float32),
    )(x, agg, cnt_part, h1, wl, bl.reshape(NB, 1, D), wr1,
      gamma.reshape(NB, 1, D), beta.reshape(NB, 1, D),
      wq, bq, wk, bk, wf, bf, wr, br)


def kernel(item_feats, edge_index, edge_type, params):
    src = edge_index[0].astype(jnp.int32)
    dst = edge_index[1].astype(jnp.int32)
    t = edge_type.astype(jnp.int32)

    pad = EP - E
    zpad = jnp.zeros((pad,), jnp.int32)
    g0 = jnp.concatenate([4 * src, zpad])
    g1 = jnp.concatenate([4 * (t * N + src), zpad])
    sd = jnp.concatenate([t * N + dst, jnp.full((pad,), DUMP, jnp.int32)])
    g0_q = jnp.stack([g0, g0 + 1, g0 + 2, g0 + 3]).reshape(4, EROWS, 128)
    g1_q = jnp.stack([g1, g1 + 1, g1 + 2, g1 + 3]).reshape(4, EROWS, 128)
    sd = sd.reshape(EROWS, 128)

    zeros32 = jnp.zeros((128, 32), jnp.float32)
    zeros16 = jnp.zeros((128, 16), jnp.float32)
    ones16 = jnp.ones((128, 16), jnp.float32)

    p = params
    emb_all = jnp.concatenate([p['user_emb'], item_feats], axis=0)
    w_io = jnp.stack([p['user_proj_W'], p['item_proj_W']])
    b_io = jnp.stack([p['user_proj_b'], p['item_proj_b']])

    wl = [jnp.stack([p['block%d_layer%d' % (b, l)]['Wl'] for b in range(NB)])
          for l in range(2)]
    bl = [jnp.stack([p['block%d_layer%d' % (b, l)]['bl'] for b in range(NB)])
          for l in range(2)]
    wr = [jnp.stack([p['block%d_layer%d' % (b, l)]['Wr'] for b in range(NB)])
          for l in range(2)]
    gm = [jnp.stack([p['block%d_layer%d' % (b, l)]['bn_gamma'] for b in range(NB)])
          for l in range(2)]
    bt = [jnp.stack([p['block%d_layer%d' % (b, l)]['bn_beta'] for b in range(NB)])
          for l in range(2)]
    wk = jnp.stack([p['key_proj%d_W' % b] for b in range(NB)])
    bk = jnp.stack([p['key_proj%d_b' % b] for b in range(NB)])

    x = _tc_proj(emb_all, w_io, b_io)

    cnt_part = _sc_count(sd, ones16, zeros16)
    cnt4 = cnt_part.reshape(2, NB, N, 16)

    agg0h = _sc_segsum(x.reshape(4 * N, 32), g0_q, sd, zeros32)
    h1 = _tc_post(agg0h.reshape(4, NB, N, 32), cnt4, x,
                  wl[0], bl[0], wr[0], gm[0], bt[0], True)

    agg1h = _sc_segsum(h1.reshape(4 * T3, 32), g1_q, sd, zeros32)

    return _tc_fuse(x, agg1h.reshape(4, NB, N, 32), cnt4, h1,
                    wl[1], bl[1], wr[1], gm[1], bt[1],
                    p['query_proj_W'], p['query_proj_b'].reshape(1, D),
                    wk, bk, p['fuse_W'], p['fuse_b'].reshape(1, D),
                    p['refine_W'], p['refine_b'].reshape(1, D))
